# baseline (speedup 1.0000x reference)
import functools

import jax
import jax.numpy as jnp
from jax import lax
from jax.experimental import pallas as pl
from jax.experimental.pallas import tpu as pltpu
from jax.experimental.pallas import tpu_sc as plsc

VOCAB = 1000000
EMBED_DIM = 64
BATCH = 4096
SEQ = 200

NUM_CORES = 2
NUM_SUBCORES = 16
NW = NUM_CORES * NUM_SUBCORES
ROWS_W = BATCH // NW
SLAB = 2
STEPS = ROWS_W // SLAB
PIECES = [(0, 128), (128, 72)]
NBUF = 2

_mesh = plsc.VectorSubcoreMesh(core_axis_name="c", subcore_axis_name="s")


@functools.partial(
    pl.kernel,
    out_type=jax.ShapeDtypeStruct((BATCH, SEQ, EMBED_DIM), jnp.float32),
    mesh=_mesh,
    scratch_types=[
        pltpu.VMEM((STEPS, SLAB * SEQ), jnp.int32),
        pltpu.VMEM((NBUF, SLAB, SEQ, EMBED_DIM), jnp.float32),
        pltpu.SemaphoreType.DMA((NBUF,)),
        pltpu.SemaphoreType.DMA((NBUF,)),
    ],
    compiler_params=pltpu.CompilerParams(use_tc_tiling_on_sc=False),
)
def _sc_gather(idx_hbm, table_hbm, out_hbm, idx_v, rows_v, gsem, osem):
    wid = lax.axis_index("s") * NUM_CORES + lax.axis_index("c")
    b0 = wid * ROWS_W
    pltpu.sync_copy(idx_hbm.at[wid], idx_v)

    def fire_gathers(s, buf):
        for r in range(SLAB):
            for off, n in PIECES:
                pltpu.async_copy(
                    table_hbm.at[idx_v.at[s, pl.ds(r * SEQ + off, n)]],
                    rows_v.at[buf, r, pl.ds(off, n)], gsem.at[buf])

    def wait_gathers(buf):
        for r in range(SLAB):
            for off, n in PIECES:
                pltpu.make_async_copy(
                    table_hbm.at[pl.ds(0, n)],
                    rows_v.at[buf, r, pl.ds(off, n)], gsem.at[buf]).wait()

    def wait_store(buf):
        pltpu.make_async_copy(
            rows_v.at[buf], out_hbm.at[pl.ds(0, SLAB)], osem.at[buf]).wait()

    fire_gathers(0, 0)

    def superstep(g, carry):
        for sub in range(NBUF):
            s = g * NBUF + sub
            nb = (sub + 1) % NBUF

            @pl.when(s + 1 < STEPS)
            def _():
                @pl.when(s + 1 >= NBUF)
                def _():
                    wait_store(nb)

                fire_gathers(s + 1, nb)

            @pl.when(s < STEPS)
            def _():
                wait_gathers(sub)
                pltpu.async_copy(
                    rows_v.at[sub],
                    out_hbm.at[pl.ds(b0 + s * SLAB, SLAB)], osem.at[sub])
        return carry

    lax.fori_loop(0, STEPS // NBUF, superstep, 0)
    for b in range(NBUF):
        wait_store(b)


def kernel(x, embedding_table):
    idx = x.reshape(NW, STEPS, SLAB * SEQ).astype(jnp.int32)
    return _sc_gather(idx, embedding_table)
